# Initial kernel scaffold; baseline (speedup 1.0000x reference)
import jax
import jax.numpy as jnp
from jax.experimental import pallas as pl


def kernel(input, lut, bias, input_mask):
    raise NotImplementedError("write your pallas kernel here")


# speedup vs baseline: 5.8806x; 5.8806x over previous
import functools

import jax
import jax.numpy as jnp
from jax import lax
from jax.experimental import pallas as pl
from jax.experimental.pallas import tpu as pltpu
from jax.experimental.pallas import tpu_sc as plsc

IN_F = 128
OUT_F = 64
BATCH = 1024
LANES = 16
NC = 2
NS = 16
NW = NC * NS
BPW = BATCH // NW
OG = OUT_F // LANES


def _coef_body(l0, l1, l2, l3, bias, c10, c01, c11, b2):
    a0 = l0[...]
    a1 = l1[...]
    a2 = l2[...]
    a3 = l3[...]
    c10[...] = a1 - a0
    c01[...] = a2 - a0
    c11[...] = (a0 - a1) + (a3 - a2)
    b2[...] = bias[...] + jnp.sum(a0, axis=0)[None, :]


_coef_call = pl.pallas_call(
    _coef_body,
    out_shape=(
        jax.ShapeDtypeStruct((IN_F, OUT_F), jnp.float32),
        jax.ShapeDtypeStruct((IN_F, OUT_F), jnp.float32),
        jax.ShapeDtypeStruct((IN_F, OUT_F), jnp.float32),
        jax.ShapeDtypeStruct((1, OUT_F), jnp.float32),
    ),
)


_sc_mesh = plsc.VectorSubcoreMesh(core_axis_name="c", subcore_axis_name="s")


@functools.partial(
    pl.kernel,
    out_type=jax.ShapeDtypeStruct((BATCH, OUT_F), jnp.float32),
    mesh=_sc_mesh,
    compiler_params=pltpu.CompilerParams(needs_layout_passes=False),
    scratch_types=[
        pltpu.VMEM((BPW * IN_F,), jnp.float32),
        pltpu.VMEM((3, IN_F, OUT_F), jnp.float32),
        pltpu.VMEM((IN_F, OUT_F), jnp.int32),
        pltpu.VMEM((OUT_F,), jnp.float32),
        pltpu.VMEM((BPW, OUT_F), jnp.float32),
    ],
)
def _sc_kernel(x_hbm, coef_hbm, idx_hbm, b2_hbm, out_hbm, xv, cf, ix, b2, ov):
    wid = lax.axis_index("s") * NC + lax.axis_index("c")
    base = wid * BPW
    pltpu.sync_copy(x_hbm.at[pl.ds(base * IN_F, BPW * IN_F)], xv)
    pltpu.sync_copy(coef_hbm, cf)
    pltpu.sync_copy(idx_hbm, ix)
    pltpu.sync_copy(b2_hbm, b2)

    for og in range(OG):
        sl = pl.ds(og * LANES, LANES)
        bv = b2[sl]

        @plsc.parallel_loop(0, BPW, unroll=4)
        def init_b(b, sl=sl, bv=bv):
            ov[b, sl] = bv

    def body_i(i, _):
        ii = jnp.full((LANES,), i, jnp.int32)
        regs = []
        for og in range(OG):
            sl = pl.ds(og * LANES, LANES)
            regs.append((cf[0, i, sl], cf[1, i, sl], cf[2, i, sl], ix[i, sl]))

        @plsc.parallel_loop(0, BPW, unroll=4)
        def body_b(b):
            bb = jnp.full((LANES,), b * IN_F, jnp.int32)
            e0 = plsc.load_gather(xv, [bb + ii])
            for og in range(OG):
                c10, c01, c11, iv = regs[og]
                e1 = plsc.load_gather(xv, [bb + iv])
                val = c10 * e0 + e1 * (c01 + c11 * e0)
                plsc.addupdate(ov.at[b, pl.ds(og * LANES, LANES)], val)

        return 0

    lax.fori_loop(0, IN_F, body_i, 0)
    pltpu.sync_copy(ov, out_hbm.at[pl.ds(base, BPW)])


def kernel(input, lut, bias, input_mask):
    lt = lut.reshape(OUT_F, IN_F, 4).transpose(1, 0, 2)
    l0, l1, l2, l3 = (lt[:, :, a] for a in range(4))
    c10, c01, c11, b2 = _coef_call(l0, l1, l2, l3, bias.reshape(1, OUT_F))
    coef = jnp.stack([c10, c01, c11], axis=0)
    idx_t = input_mask.reshape(OUT_F, IN_F, 2)[:, :, 1].transpose()
    return _sc_kernel(input.reshape(-1), coef, idx_t, b2.reshape(OUT_F))
